# Initial kernel scaffold; baseline (speedup 1.0000x reference)
import jax
import jax.numpy as jnp
from jax.experimental import pallas as pl


def kernel(x_nchw, w1, b1, w2, b2):
    raise NotImplementedError("write your pallas kernel here")


# speedup vs baseline: 2.6887x; 2.6887x over previous
import functools

import jax
import jax.numpy as jnp
from jax.experimental import pallas as pl
from jax.experimental.pallas import tpu as pltpu

_SLOPE = 0.05


def _se_slab_kernel(x_ref, w1_ref, b1_ref, w2_ref, b2_ref, o_ref, *, inv_hw):
    x = x_ref[0]
    pooled = jnp.sum(x, axis=1, keepdims=True) * inv_hw
    h = jnp.dot(w1_ref[...], pooled,
                preferred_element_type=jnp.float32) + b1_ref[...]
    h = jnp.maximum(h, 0.0) + _SLOPE * jnp.minimum(h, 0.0)
    g = jnp.dot(w2_ref[...], h,
                preferred_element_type=jnp.float32) + b2_ref[...]
    gate = jax.nn.sigmoid(g)
    o_ref[0] = x * gate


def kernel(x_nchw, w1, b1, w2, b2):
    B, C, H, W = x_nchw.shape
    HW = H * W
    x3 = x_nchw.reshape(B, C, HW)

    slab_bytes = C * pl.cdiv(HW, 128) * 128 * x3.dtype.itemsize
    vmem = 4 * slab_bytes + 2 * C * C * 4 + (4 << 20)

    out = pl.pallas_call(
        functools.partial(_se_slab_kernel, inv_hw=1.0 / HW),
        out_shape=jax.ShapeDtypeStruct((B, C, HW), x3.dtype),
        grid=(B,),
        in_specs=[
            pl.BlockSpec((1, C, HW), lambda b: (b, 0, 0)),
            pl.BlockSpec((C, C), lambda b: (0, 0)),
            pl.BlockSpec((C, 1), lambda b: (0, 0)),
            pl.BlockSpec((C, C), lambda b: (0, 0)),
            pl.BlockSpec((C, 1), lambda b: (0, 0)),
        ],
        out_specs=pl.BlockSpec((1, C, HW), lambda b: (b, 0, 0)),
        compiler_params=pltpu.CompilerParams(
            dimension_semantics=("parallel",),
            vmem_limit_bytes=int(vmem)),
    )(x3, w1, b1.reshape(C, 1), w2, b2.reshape(C, 1))

    return out.reshape(B, C, H, W)


# speedup vs baseline: 2.7552x; 1.0247x over previous
import functools

import jax
import jax.numpy as jnp
from jax.experimental import pallas as pl
from jax.experimental.pallas import tpu as pltpu

_SLOPE = 0.05


def _se_group_kernel(x_ref, w1t_ref, b1_ref, w2t_ref, b2_ref, o_ref, *,
                     inv_hw):
    x = x_ref[...]
    pooled = jnp.sum(x, axis=2) * inv_hw
    h = jnp.dot(pooled, w1t_ref[...],
                preferred_element_type=jnp.float32) + b1_ref[...]
    h = jnp.maximum(h, 0.0) + _SLOPE * jnp.minimum(h, 0.0)
    g = jnp.dot(h, w2t_ref[...],
                preferred_element_type=jnp.float32) + b2_ref[...]
    s = jax.nn.sigmoid(g)
    o_ref[...] = x * s[:, :, None]


def kernel(x_nchw, w1, b1, w2, b2):
    B, C, H, W = x_nchw.shape
    HW = H * W
    x3 = x_nchw.reshape(B, C, HW)

    slab_bytes = C * pl.cdiv(HW, 128) * 128 * x3.dtype.itemsize
    group = max(1, min(B, (52 << 20) // (4 * slab_bytes)))
    while B % group:
        group -= 1
    n_steps = B // group

    vmem = 4 * group * slab_bytes + 2 * C * C * 4 + (4 << 20)

    out = pl.pallas_call(
        functools.partial(_se_group_kernel, inv_hw=1.0 / HW),
        out_shape=jax.ShapeDtypeStruct((B, C, HW), x3.dtype),
        grid=(n_steps,),
        in_specs=[
            pl.BlockSpec((group, C, HW), lambda i: (i, 0, 0)),
            pl.BlockSpec((C, C), lambda i: (0, 0)),
            pl.BlockSpec((1, C), lambda i: (0, 0)),
            pl.BlockSpec((C, C), lambda i: (0, 0)),
            pl.BlockSpec((1, C), lambda i: (0, 0)),
        ],
        out_specs=pl.BlockSpec((group, C, HW), lambda i: (i, 0, 0)),
        compiler_params=pltpu.CompilerParams(
            dimension_semantics=("arbitrary",),
            vmem_limit_bytes=int(min(vmem, 60 << 20))),
    )(x3, w1.T, b1.reshape(1, C), w2.T, b2.reshape(1, C))

    return out.reshape(B, C, H, W)
